# Initial kernel scaffold; baseline (speedup 1.0000x reference)
import jax
import jax.numpy as jnp
from jax.experimental import pallas as pl


def kernel(x, Wg, W1, b1, W2, b2):
    raise NotImplementedError("write your pallas kernel here")


# speedup vs baseline: 1.2933x; 1.2933x over previous
import functools

import jax
import jax.numpy as jnp
from jax import lax
from jax.experimental import pallas as pl
from jax.experimental.pallas import tpu as pltpu
from jax.experimental.pallas import tpu_sc as plsc

E = 64
D = 1024
FF = 1024
T = 8192
CAP = 160
CHUNK = 1024
NROWS = E * CAP
XE_ROWS = NROWS + 8
WIN = 32


def _router_body(x_ref, wg_ref, idxd_ref, idxc_ref, gk_ref, cnt_ref):
    step = pl.program_id(0)

    @pl.when(step == 0)
    def _():
        cnt_ref[...] = jnp.zeros_like(cnt_ref)

    xb = x_ref[...]
    wg = wg_ref[...]
    logits = jnp.dot(xb, wg, preferred_element_type=jnp.float32,
                     precision=lax.Precision.HIGHEST)
    m = jnp.max(logits, axis=1, keepdims=True)
    ssum = jnp.sum(jnp.exp(logits - m), axis=1, keepdims=True)
    g = 1.0 / ssum

    lane = lax.broadcasted_iota(jnp.int32, (CHUNK, E), 1)
    e_idx = jnp.min(jnp.where(logits == m, lane, E), axis=1, keepdims=True)
    onehot = (lane == e_idx).astype(jnp.float32)

    r = lax.broadcasted_iota(jnp.int32, (CHUNK, CHUNK), 0)
    c = lax.broadcasted_iota(jnp.int32, (CHUNK, CHUNK), 1)
    tri = (c < r).astype(jnp.bfloat16)
    pos_in = jnp.dot(tri, onehot.astype(jnp.bfloat16),
                     preferred_element_type=jnp.float32)

    carry = cnt_ref[0:1, :]
    pos = jnp.sum((pos_in + carry) * onehot, axis=1, keepdims=True)
    cnt_ref[0:1, :] = carry + jnp.sum(onehot, axis=0, keepdims=True)

    pos_i = pos.astype(jnp.int32)
    keep = pos_i < CAP
    slot = e_idx * CAP + jnp.minimum(pos_i, CAP - 1)
    idxd_ref[...] = jnp.where(keep, slot, NROWS)
    idxc_ref[...] = jnp.where(keep, slot, 0)
    gk_ref[...] = jnp.where(keep, g, 0.0)


def _router(x, wg):
    return pl.pallas_call(
        _router_body,
        grid=(T // CHUNK,),
        in_specs=[
            pl.BlockSpec((CHUNK, D), lambda i: (i, 0)),
            pl.BlockSpec((D, E), lambda i: (0, 0)),
        ],
        out_specs=[
            pl.BlockSpec((CHUNK, 1), lambda i: (i, 0)),
            pl.BlockSpec((CHUNK, 1), lambda i: (i, 0)),
            pl.BlockSpec((CHUNK, 1), lambda i: (i, 0)),
        ],
        out_shape=[
            jax.ShapeDtypeStruct((T, 1), jnp.int32),
            jax.ShapeDtypeStruct((T, 1), jnp.int32),
            jax.ShapeDtypeStruct((T, 1), jnp.float32),
        ],
        scratch_shapes=[pltpu.VMEM((8, E), jnp.float32)],
    )(x, wg)


NC = 2
NS = 16
NW = NC * NS
CPT = T // NW
NCHUNK = CPT // WIN


def _dispatch(x, idx):
    idx2 = idx.reshape(T // WIN, WIN)
    mesh = plsc.VectorSubcoreMesh(core_axis_name="core",
                                  subcore_axis_name="subcore")

    @functools.partial(
        pl.kernel,
        out_type=jax.ShapeDtypeStruct((XE_ROWS, D), jnp.float32),
        mesh=mesh,
        scratch_types=[
            pltpu.VMEM((NCHUNK, WIN), jnp.int32),
            pltpu.VMEM((WIN, D), jnp.float32),
            pltpu.SemaphoreType.DMA,
        ])
    def k(x_hbm, i_hbm, o_hbm, idx_v, buf, sem):
        wid = lax.axis_index("subcore") * NC + lax.axis_index("core")
        base = wid * CPT
        pltpu.sync_copy(i_hbm.at[pl.ds(wid * NCHUNK, NCHUNK)], idx_v)
        for j in range(NCHUNK):
            pltpu.sync_copy(x_hbm.at[pl.ds(base + j * WIN, WIN)], buf)
            pltpu.async_copy(buf, o_hbm.at[idx_v.at[j]], sem).wait()

    return k(x, idx2)


def _ffn_body(xe_ref, w1_ref, b1_ref, w2_ref, b2_ref, ye_ref):
    xb = xe_ref[...].astype(jnp.bfloat16)
    w1 = w1_ref[0].astype(jnp.bfloat16)
    h = jnp.maximum(
        jnp.dot(xb, w1, preferred_element_type=jnp.float32) + b1_ref[0], 0.0)
    w2 = w2_ref[0].astype(jnp.bfloat16)
    y = jnp.dot(h.astype(jnp.bfloat16), w2,
                preferred_element_type=jnp.float32) + b2_ref[0]
    ye_ref[...] = y


def _ffn(xe, w1, b1, w2, b2):
    return pl.pallas_call(
        _ffn_body,
        grid=(E,),
        in_specs=[
            pl.BlockSpec((CAP, D), lambda e: (e, 0)),
            pl.BlockSpec((1, D, FF), lambda e: (e, 0, 0)),
            pl.BlockSpec((1, 1, FF), lambda e: (e, 0, 0)),
            pl.BlockSpec((1, FF, D), lambda e: (e, 0, 0)),
            pl.BlockSpec((1, 1, D), lambda e: (e, 0, 0)),
        ],
        out_specs=pl.BlockSpec((CAP, D), lambda e: (e, 0)),
        out_shape=jax.ShapeDtypeStruct((NROWS, D), jnp.float32),
    )(xe, w1, b1, w2, b2)


def _combine(ye, idx):
    idx2 = idx.reshape(T // WIN, WIN)
    mesh = plsc.VectorSubcoreMesh(core_axis_name="core",
                                  subcore_axis_name="subcore")

    @functools.partial(
        pl.kernel,
        out_type=jax.ShapeDtypeStruct((T, D), jnp.float32),
        mesh=mesh,
        scratch_types=[
            pltpu.VMEM((NCHUNK, WIN), jnp.int32),
            pltpu.VMEM((WIN, D), jnp.float32),
            pltpu.SemaphoreType.DMA,
        ])
    def k(ye_hbm, i_hbm, o_hbm, idx_v, buf, sem):
        wid = lax.axis_index("subcore") * NC + lax.axis_index("core")
        base = wid * CPT
        pltpu.sync_copy(i_hbm.at[pl.ds(wid * NCHUNK, NCHUNK)], idx_v)
        for j in range(NCHUNK):
            pltpu.async_copy(ye_hbm.at[idx_v.at[j]], buf, sem).wait()
            pltpu.sync_copy(buf, o_hbm.at[pl.ds(base + j * WIN, WIN)])

    return k(ye, idx2)


def _final_body(x_ref, yg_ref, gk_ref, y_ref):
    gk = gk_ref[...]
    y_ref[...] = x_ref[...] + jnp.where(gk > 0.0, gk * yg_ref[...], 0.0)


def _final(x, yg, gk):
    return pl.pallas_call(
        _final_body,
        grid=(T // CHUNK,),
        in_specs=[
            pl.BlockSpec((CHUNK, D), lambda i: (i, 0)),
            pl.BlockSpec((CHUNK, D), lambda i: (i, 0)),
            pl.BlockSpec((CHUNK, 1), lambda i: (i, 0)),
        ],
        out_specs=pl.BlockSpec((CHUNK, D), lambda i: (i, 0)),
        out_shape=jax.ShapeDtypeStruct((T, D), jnp.float32),
    )(x, yg, gk)


def kernel(x, Wg, W1, b1, W2, b2):
    idxd, idxc, gk = _router(x, Wg)
    xe = _dispatch(x, idxd.reshape(T))
    ye = _ffn(xe, W1, b1.reshape(E, 1, FF), W2, b2.reshape(E, 1, D))
    yg = _combine(ye, idxc.reshape(T))
    return _final(x, yg, gk)


# speedup vs baseline: 1.3738x; 1.0623x over previous
import functools

import jax
import jax.numpy as jnp
from jax import lax
from jax.experimental import pallas as pl
from jax.experimental.pallas import tpu as pltpu
from jax.experimental.pallas import tpu_sc as plsc

E = 64
D = 1024
FF = 1024
T = 8192
CAP = 160
CHUNK = 1024
NROWS = E * CAP
XE_ROWS = NROWS + 8
WIN = 32


def _router_body(x_ref, wg_ref, idxd_ref, idxc_ref, gk_ref, cnt_ref):
    step = pl.program_id(0)

    @pl.when(step == 0)
    def _():
        cnt_ref[...] = jnp.zeros_like(cnt_ref)

    xb = x_ref[...]
    wg = wg_ref[...]
    logits = jnp.dot(xb.astype(jnp.bfloat16), wg.astype(jnp.bfloat16),
                     preferred_element_type=jnp.float32)
    m = jnp.max(logits, axis=1, keepdims=True)
    ssum = jnp.sum(jnp.exp(logits - m), axis=1, keepdims=True)
    g = 1.0 / ssum

    lane = lax.broadcasted_iota(jnp.int32, (CHUNK, E), 1)
    e_idx = jnp.min(jnp.where(logits == m, lane, E), axis=1, keepdims=True)
    onehot = (lane == e_idx).astype(jnp.float32)

    r = lax.broadcasted_iota(jnp.int32, (CHUNK, CHUNK), 0)
    c = lax.broadcasted_iota(jnp.int32, (CHUNK, CHUNK), 1)
    tri = (c < r).astype(jnp.bfloat16)
    pos_in = jnp.dot(tri, onehot.astype(jnp.bfloat16),
                     preferred_element_type=jnp.float32)

    carry = cnt_ref[0:1, :]
    pos = jnp.sum((pos_in + carry) * onehot, axis=1, keepdims=True)
    cnt_ref[0:1, :] = carry + jnp.sum(onehot, axis=0, keepdims=True)

    pos_i = pos.astype(jnp.int32)
    keep = pos_i < CAP
    slot = e_idx * CAP + jnp.minimum(pos_i, CAP - 1)
    idxd_ref[...] = jnp.where(keep, slot, NROWS)
    idxc_ref[...] = jnp.where(keep, slot, 0)
    gk_ref[...] = jnp.where(keep, g, 0.0)


def _router(x, wg):
    return pl.pallas_call(
        _router_body,
        grid=(T // CHUNK,),
        in_specs=[
            pl.BlockSpec((CHUNK, D), lambda i: (i, 0)),
            pl.BlockSpec((D, E), lambda i: (0, 0)),
        ],
        out_specs=[
            pl.BlockSpec((CHUNK, 1), lambda i: (i, 0)),
            pl.BlockSpec((CHUNK, 1), lambda i: (i, 0)),
            pl.BlockSpec((CHUNK, 1), lambda i: (i, 0)),
        ],
        out_shape=[
            jax.ShapeDtypeStruct((T, 1), jnp.int32),
            jax.ShapeDtypeStruct((T, 1), jnp.int32),
            jax.ShapeDtypeStruct((T, 1), jnp.float32),
        ],
        scratch_shapes=[pltpu.VMEM((8, E), jnp.float32)],
    )(x, wg)


NC = 2
NS = 16
NW = NC * NS
CPT = T // NW
NCHUNK = CPT // WIN


def _dispatch(x, idx):
    idx2 = idx.reshape(T // WIN, WIN)
    mesh = plsc.VectorSubcoreMesh(core_axis_name="core",
                                  subcore_axis_name="subcore")

    @functools.partial(
        pl.kernel,
        out_type=jax.ShapeDtypeStruct((XE_ROWS, D), jnp.float32),
        mesh=mesh,
        scratch_types=[
            pltpu.VMEM((NCHUNK, WIN), jnp.int32),
            pltpu.VMEM((WIN, D), jnp.float32),
            pltpu.SemaphoreType.DMA,
        ])
    def k(x_hbm, i_hbm, o_hbm, idx_v, buf, sem):
        wid = lax.axis_index("subcore") * NC + lax.axis_index("core")
        base = wid * CPT
        pltpu.sync_copy(i_hbm.at[pl.ds(wid * NCHUNK, NCHUNK)], idx_v)
        for j in range(NCHUNK):
            pltpu.sync_copy(x_hbm.at[pl.ds(base + j * WIN, WIN)], buf)
            pltpu.async_copy(buf, o_hbm.at[idx_v.at[j]], sem).wait()

    return k(x, idx2)


def _ffn_body(xe_ref, w1_ref, b1_ref, w2_ref, b2_ref, ye_ref):
    xb = xe_ref[...].astype(jnp.bfloat16)
    w1 = w1_ref[0].astype(jnp.bfloat16)
    h = jnp.maximum(
        jnp.dot(xb, w1, preferred_element_type=jnp.float32) + b1_ref[0], 0.0)
    w2 = w2_ref[0].astype(jnp.bfloat16)
    y = jnp.dot(h.astype(jnp.bfloat16), w2,
                preferred_element_type=jnp.float32) + b2_ref[0]
    ye_ref[...] = y


def _ffn(xe, w1, b1, w2, b2):
    return pl.pallas_call(
        _ffn_body,
        grid=(E,),
        in_specs=[
            pl.BlockSpec((CAP, D), lambda e: (e, 0)),
            pl.BlockSpec((1, D, FF), lambda e: (e, 0, 0)),
            pl.BlockSpec((1, 1, FF), lambda e: (e, 0, 0)),
            pl.BlockSpec((1, FF, D), lambda e: (e, 0, 0)),
            pl.BlockSpec((1, 1, D), lambda e: (e, 0, 0)),
        ],
        out_specs=pl.BlockSpec((CAP, D), lambda e: (e, 0)),
        out_shape=jax.ShapeDtypeStruct((NROWS, D), jnp.float32),
    )(xe, w1, b1, w2, b2)


def _combine(ye, idx):
    idx2 = idx.reshape(T // WIN, WIN)
    mesh = plsc.VectorSubcoreMesh(core_axis_name="core",
                                  subcore_axis_name="subcore")

    @functools.partial(
        pl.kernel,
        out_type=jax.ShapeDtypeStruct((T, D), jnp.float32),
        mesh=mesh,
        scratch_types=[
            pltpu.VMEM((NCHUNK, WIN), jnp.int32),
            pltpu.VMEM((WIN, D), jnp.float32),
            pltpu.SemaphoreType.DMA,
        ])
    def k(ye_hbm, i_hbm, o_hbm, idx_v, buf, sem):
        wid = lax.axis_index("subcore") * NC + lax.axis_index("core")
        base = wid * CPT
        pltpu.sync_copy(i_hbm.at[pl.ds(wid * NCHUNK, NCHUNK)], idx_v)
        for j in range(NCHUNK):
            pltpu.async_copy(ye_hbm.at[idx_v.at[j]], buf, sem).wait()
            pltpu.sync_copy(buf, o_hbm.at[pl.ds(base + j * WIN, WIN)])

    return k(ye, idx2)


def _final_body(x_ref, yg_ref, gk_ref, y_ref):
    gk = gk_ref[...]
    y_ref[...] = x_ref[...] + jnp.where(gk > 0.0, gk * yg_ref[...], 0.0)


def _final(x, yg, gk):
    return pl.pallas_call(
        _final_body,
        grid=(T // CHUNK,),
        in_specs=[
            pl.BlockSpec((CHUNK, D), lambda i: (i, 0)),
            pl.BlockSpec((CHUNK, D), lambda i: (i, 0)),
            pl.BlockSpec((CHUNK, 1), lambda i: (i, 0)),
        ],
        out_specs=pl.BlockSpec((CHUNK, D), lambda i: (i, 0)),
        out_shape=jax.ShapeDtypeStruct((T, D), jnp.float32),
    )(x, yg, gk)


def kernel(x, Wg, W1, b1, W2, b2):
    idxd, idxc, gk = _router(x, Wg)
    xe = _dispatch(x, idxd.reshape(T))
    ye = _ffn(xe, W1, b1.reshape(E, 1, FF), W2, b2.reshape(E, 1, D))
    yg = _combine(ye, idxc.reshape(T))
    return _final(x, yg, gk)


# speedup vs baseline: 1.5451x; 1.1247x over previous
import functools

import jax
import jax.numpy as jnp
from jax import lax
from jax.experimental import pallas as pl
from jax.experimental.pallas import tpu as pltpu
from jax.experimental.pallas import tpu_sc as plsc

E = 64
D = 1024
FF = 1024
T = 8192
CAP = 160
CHUNK = 1024
NROWS = E * CAP
XE_ROWS = NROWS + 8
WIN = 32


def _router_body(x_ref, wg_ref, idxd_ref, idxc_ref, gk_ref, xb_ref, cnt_ref):
    step = pl.program_id(0)

    @pl.when(step == 0)
    def _():
        cnt_ref[...] = jnp.zeros_like(cnt_ref)

    xb = x_ref[...].astype(jnp.bfloat16)
    xb_ref[...] = _pack2(xb[:, :DP], xb[:, DP:])
    wg = wg_ref[...]
    logits = jnp.dot(xb, wg.astype(jnp.bfloat16),
                     preferred_element_type=jnp.float32)
    m = jnp.max(logits, axis=1, keepdims=True)
    ssum = jnp.sum(jnp.exp(logits - m), axis=1, keepdims=True)
    g = 1.0 / ssum

    lane = lax.broadcasted_iota(jnp.int32, (CHUNK, E), 1)
    e_idx = jnp.min(jnp.where(logits == m, lane, E), axis=1, keepdims=True)
    onehot = (lane == e_idx).astype(jnp.float32)

    r = lax.broadcasted_iota(jnp.int32, (CHUNK, CHUNK), 0)
    c = lax.broadcasted_iota(jnp.int32, (CHUNK, CHUNK), 1)
    tri = (c < r).astype(jnp.bfloat16)
    pos_in = jnp.dot(tri, onehot.astype(jnp.bfloat16),
                     preferred_element_type=jnp.float32)

    carry = cnt_ref[0:1, :]
    pos = jnp.sum((pos_in + carry) * onehot, axis=1, keepdims=True)
    cnt_ref[0:1, :] = carry + jnp.sum(onehot, axis=0, keepdims=True)

    pos_i = pos.astype(jnp.int32)
    keep = pos_i < CAP
    slot = e_idx * CAP + jnp.minimum(pos_i, CAP - 1)
    idxd_ref[...] = jnp.where(keep, slot, NROWS)
    idxc_ref[...] = jnp.where(keep, slot, 0)
    gk_ref[...] = jnp.where(keep, g, 0.0)


def _router(x, wg):
    return pl.pallas_call(
        _router_body,
        grid=(T // CHUNK,),
        in_specs=[
            pl.BlockSpec((CHUNK, D), lambda i: (i, 0)),
            pl.BlockSpec((D, E), lambda i: (0, 0)),
        ],
        out_specs=[
            pl.BlockSpec((CHUNK, 1), lambda i: (i, 0)),
            pl.BlockSpec((CHUNK, 1), lambda i: (i, 0)),
            pl.BlockSpec((CHUNK, 1), lambda i: (i, 0)),
            pl.BlockSpec((CHUNK, DP), lambda i: (i, 0)),
        ],
        out_shape=[
            jax.ShapeDtypeStruct((T, 1), jnp.int32),
            jax.ShapeDtypeStruct((T, 1), jnp.int32),
            jax.ShapeDtypeStruct((T, 1), jnp.float32),
            jax.ShapeDtypeStruct((T, DP), jnp.float32),
        ],
        scratch_shapes=[pltpu.VMEM((8, E), jnp.float32)],
    )(x, wg)


DP = D // 2


def _pack2(a, b):
    au = lax.bitcast_convert_type(a, jnp.uint16).astype(jnp.uint32)
    bu = lax.bitcast_convert_type(b, jnp.uint16).astype(jnp.uint32)
    return lax.bitcast_convert_type((au << 16) | bu, jnp.float32)


def _unpack2(w):
    wu = lax.bitcast_convert_type(w, jnp.uint32)
    a = lax.bitcast_convert_type((wu >> 16).astype(jnp.uint16), jnp.bfloat16)
    b = lax.bitcast_convert_type(wu.astype(jnp.uint16), jnp.bfloat16)
    return a, b


NC = 2
NS = 16
NW = NC * NS
CPT = T // NW
NCHUNK = CPT // WIN


def _dispatch(x, idx):
    idx2 = idx.reshape(T // WIN, WIN)
    mesh = plsc.VectorSubcoreMesh(core_axis_name="core",
                                  subcore_axis_name="subcore")

    @functools.partial(
        pl.kernel,
        out_type=jax.ShapeDtypeStruct((XE_ROWS, DP), jnp.float32),
        mesh=mesh,
        scratch_types=[
            pltpu.VMEM((NCHUNK, WIN), jnp.int32),
            pltpu.VMEM((WIN, DP), jnp.float32),
            pltpu.SemaphoreType.DMA,
        ])
    def k(x_hbm, i_hbm, o_hbm, idx_v, buf, sem):
        wid = lax.axis_index("subcore") * NC + lax.axis_index("core")
        base = wid * CPT
        pltpu.sync_copy(i_hbm.at[pl.ds(wid * NCHUNK, NCHUNK)], idx_v)
        for j in range(NCHUNK):
            pltpu.sync_copy(x_hbm.at[pl.ds(base + j * WIN, WIN)], buf)
            pltpu.async_copy(buf, o_hbm.at[idx_v.at[j]], sem).wait()

    return k(x, idx2)


def _ffn_body(xe_ref, w1_ref, b1_ref, w2_ref, b2_ref, ye_ref):
    xa, xc = _unpack2(xe_ref[...])
    xb = jnp.concatenate([xa, xc], axis=1)
    w1 = w1_ref[0].astype(jnp.bfloat16)
    h = jnp.maximum(
        jnp.dot(xb, w1, preferred_element_type=jnp.float32) + b1_ref[0], 0.0)
    w2 = w2_ref[0].astype(jnp.bfloat16)
    y = jnp.dot(h.astype(jnp.bfloat16), w2,
                preferred_element_type=jnp.float32) + b2_ref[0]
    yb = y.astype(jnp.bfloat16)
    ye_ref[...] = _pack2(yb[:, :DP], yb[:, DP:])


def _ffn(xe, w1, b1, w2, b2):
    return pl.pallas_call(
        _ffn_body,
        grid=(E,),
        in_specs=[
            pl.BlockSpec((CAP, DP), lambda e: (e, 0)),
            pl.BlockSpec((1, D, FF), lambda e: (e, 0, 0)),
            pl.BlockSpec((1, 1, FF), lambda e: (e, 0, 0)),
            pl.BlockSpec((1, FF, D), lambda e: (e, 0, 0)),
            pl.BlockSpec((1, 1, D), lambda e: (e, 0, 0)),
        ],
        out_specs=pl.BlockSpec((CAP, DP), lambda e: (e, 0)),
        out_shape=jax.ShapeDtypeStruct((NROWS, DP), jnp.float32),
    )(xe, w1, b1, w2, b2)


def _combine(ye, idx):
    idx2 = idx.reshape(T // WIN, WIN)
    mesh = plsc.VectorSubcoreMesh(core_axis_name="core",
                                  subcore_axis_name="subcore")

    @functools.partial(
        pl.kernel,
        out_type=jax.ShapeDtypeStruct((T, DP), jnp.float32),
        mesh=mesh,
        scratch_types=[
            pltpu.VMEM((NCHUNK, WIN), jnp.int32),
            pltpu.VMEM((WIN, DP), jnp.float32),
            pltpu.SemaphoreType.DMA,
        ])
    def k(ye_hbm, i_hbm, o_hbm, idx_v, buf, sem):
        wid = lax.axis_index("subcore") * NC + lax.axis_index("core")
        base = wid * CPT
        pltpu.sync_copy(i_hbm.at[pl.ds(wid * NCHUNK, NCHUNK)], idx_v)
        for j in range(NCHUNK):
            pltpu.async_copy(ye_hbm.at[idx_v.at[j]], buf, sem).wait()
            pltpu.sync_copy(buf, o_hbm.at[pl.ds(base + j * WIN, WIN)])

    return k(ye, idx2)


def _final_body(x_ref, yg_ref, gk_ref, y_ref):
    gk = gk_ref[...]
    ya, yb = _unpack2(yg_ref[...])
    yg = jnp.concatenate([ya, yb], axis=1).astype(jnp.float32)
    y_ref[...] = x_ref[...] + jnp.where(gk > 0.0, gk * yg, 0.0)


def _final(x, yg, gk):
    return pl.pallas_call(
        _final_body,
        grid=(T // CHUNK,),
        in_specs=[
            pl.BlockSpec((CHUNK, D), lambda i: (i, 0)),
            pl.BlockSpec((CHUNK, DP), lambda i: (i, 0)),
            pl.BlockSpec((CHUNK, 1), lambda i: (i, 0)),
        ],
        out_specs=pl.BlockSpec((CHUNK, D), lambda i: (i, 0)),
        out_shape=jax.ShapeDtypeStruct((T, D), jnp.float32),
    )(x, yg, gk)


def kernel(x, Wg, W1, b1, W2, b2):
    idxd, idxc, gk, xbf = _router(x, Wg)
    xe = _dispatch(xbf, idxd.reshape(T))
    ye = _ffn(xe, W1, b1.reshape(E, 1, FF), W2, b2.reshape(E, 1, D))
    yg = _combine(ye, idxc.reshape(T))
    return _final(x, yg, gk)


# speedup vs baseline: 1.5815x; 1.0235x over previous
import functools

import jax
import jax.numpy as jnp
from jax import lax
from jax.experimental import pallas as pl
from jax.experimental.pallas import tpu as pltpu
from jax.experimental.pallas import tpu_sc as plsc

E = 64
D = 1024
FF = 1024
T = 8192
CAP = 160
CHUNK = 1024
NROWS = E * CAP
XE_ROWS = NROWS + 8
WIN = 64


def _router_body(x_ref, wg_ref, idxd_ref, idxc_ref, gk_ref, xb_ref, cnt_ref):
    step = pl.program_id(0)

    @pl.when(step == 0)
    def _():
        cnt_ref[...] = jnp.zeros_like(cnt_ref)

    xb = x_ref[...].astype(jnp.bfloat16)
    xb_ref[...] = _pack2(xb[:, :DP], xb[:, DP:])
    wg = wg_ref[...]
    logits = jnp.dot(xb, wg.astype(jnp.bfloat16),
                     preferred_element_type=jnp.float32)
    m = jnp.max(logits, axis=1, keepdims=True)
    ssum = jnp.sum(jnp.exp(logits - m), axis=1, keepdims=True)
    g = 1.0 / ssum

    lane = lax.broadcasted_iota(jnp.int32, (CHUNK, E), 1)
    e_idx = jnp.min(jnp.where(logits == m, lane, E), axis=1, keepdims=True)
    onehot = (lane == e_idx).astype(jnp.float32)

    r = lax.broadcasted_iota(jnp.int32, (CHUNK, CHUNK), 0)
    c = lax.broadcasted_iota(jnp.int32, (CHUNK, CHUNK), 1)
    tri = (c < r).astype(jnp.bfloat16)
    pos_in = jnp.dot(tri, onehot.astype(jnp.bfloat16),
                     preferred_element_type=jnp.float32)

    carry = cnt_ref[0:1, :]
    pos = jnp.sum((pos_in + carry) * onehot, axis=1, keepdims=True)
    cnt_ref[0:1, :] = carry + jnp.sum(onehot, axis=0, keepdims=True)

    pos_i = pos.astype(jnp.int32)
    keep = pos_i < CAP
    slot = e_idx * CAP + jnp.minimum(pos_i, CAP - 1)
    idxd_ref[...] = jnp.where(keep, slot, NROWS)
    idxc_ref[...] = jnp.where(keep, slot, 0)
    gk_ref[...] = jnp.where(keep, g, 0.0)


def _router(x, wg):
    return pl.pallas_call(
        _router_body,
        grid=(T // CHUNK,),
        in_specs=[
            pl.BlockSpec((CHUNK, D), lambda i: (i, 0)),
            pl.BlockSpec((D, E), lambda i: (0, 0)),
        ],
        out_specs=[
            pl.BlockSpec((CHUNK, 1), lambda i: (i, 0)),
            pl.BlockSpec((CHUNK, 1), lambda i: (i, 0)),
            pl.BlockSpec((CHUNK, 1), lambda i: (i, 0)),
            pl.BlockSpec((CHUNK, DP), lambda i: (i, 0)),
        ],
        out_shape=[
            jax.ShapeDtypeStruct((T, 1), jnp.int32),
            jax.ShapeDtypeStruct((T, 1), jnp.int32),
            jax.ShapeDtypeStruct((T, 1), jnp.float32),
            jax.ShapeDtypeStruct((T, DP), jnp.float32),
        ],
        scratch_shapes=[pltpu.VMEM((8, E), jnp.float32)],
    )(x, wg)


DP = D // 2


def _pack2(a, b):
    au = lax.bitcast_convert_type(a, jnp.uint16).astype(jnp.uint32)
    bu = lax.bitcast_convert_type(b, jnp.uint16).astype(jnp.uint32)
    return lax.bitcast_convert_type((au << 16) | bu, jnp.float32)


def _unpack2(w):
    wu = lax.bitcast_convert_type(w, jnp.uint32)
    a = lax.bitcast_convert_type((wu >> 16).astype(jnp.uint16), jnp.bfloat16)
    b = lax.bitcast_convert_type(wu.astype(jnp.uint16), jnp.bfloat16)
    return a, b


NC = 2
NS = 16
NW = NC * NS
CPT = T // NW
NCHUNK = CPT // WIN


def _dispatch(x, idx):
    idx2 = idx.reshape(T // WIN, WIN)
    mesh = plsc.VectorSubcoreMesh(core_axis_name="core",
                                  subcore_axis_name="subcore")

    @functools.partial(
        pl.kernel,
        out_type=jax.ShapeDtypeStruct((XE_ROWS, DP), jnp.float32),
        mesh=mesh,
        scratch_types=[
            pltpu.VMEM((NCHUNK, WIN), jnp.int32),
            pltpu.VMEM((WIN, DP), jnp.float32),
            pltpu.VMEM((WIN, DP), jnp.float32),
            pltpu.SemaphoreType.DMA,
            pltpu.SemaphoreType.DMA,
            pltpu.SemaphoreType.DMA,
            pltpu.SemaphoreType.DMA,
        ])
    def k(x_hbm, i_hbm, o_hbm, idx_v, buf0, buf1, sl0, sl1, ss0, ss1):
        wid = lax.axis_index("subcore") * NC + lax.axis_index("core")
        base = wid * CPT
        bufs, sls, sss = (buf0, buf1), (sl0, sl1), (ss0, ss1)
        pltpu.sync_copy(i_hbm.at[pl.ds(wid * NCHUNK, NCHUNK)], idx_v)
        loads = [None] * NCHUNK
        scats = [None] * NCHUNK
        loads[0] = pltpu.async_copy(
            x_hbm.at[pl.ds(base, WIN)], bufs[0], sls[0])
        for j in range(NCHUNK):
            b = j & 1
            loads[j].wait()
            scats[j] = pltpu.async_copy(
                bufs[b], o_hbm.at[idx_v.at[j]], sss[b])
            if j >= 1:
                scats[j - 1].wait()
            if j + 1 < NCHUNK:
                loads[j + 1] = pltpu.async_copy(
                    x_hbm.at[pl.ds(base + (j + 1) * WIN, WIN)],
                    bufs[(j + 1) & 1], sls[(j + 1) & 1])
        scats[NCHUNK - 1].wait()

    return k(x, idx2)


def _ffn_body(xe_ref, w1_ref, b1_ref, w2_ref, b2_ref, ye_ref):
    xa, xc = _unpack2(xe_ref[...])
    xb = jnp.concatenate([xa, xc], axis=1)
    w1 = w1_ref[0].astype(jnp.bfloat16)
    h = jnp.maximum(
        jnp.dot(xb, w1, preferred_element_type=jnp.float32) + b1_ref[0], 0.0)
    w2 = w2_ref[0].astype(jnp.bfloat16)
    y = jnp.dot(h.astype(jnp.bfloat16), w2,
                preferred_element_type=jnp.float32) + b2_ref[0]
    yb = y.astype(jnp.bfloat16)
    ye_ref[...] = _pack2(yb[:, :DP], yb[:, DP:])


def _ffn(xe, w1, b1, w2, b2):
    return pl.pallas_call(
        _ffn_body,
        grid=(E,),
        in_specs=[
            pl.BlockSpec((CAP, DP), lambda e: (e, 0)),
            pl.BlockSpec((1, D, FF), lambda e: (e, 0, 0)),
            pl.BlockSpec((1, 1, FF), lambda e: (e, 0, 0)),
            pl.BlockSpec((1, FF, D), lambda e: (e, 0, 0)),
            pl.BlockSpec((1, 1, D), lambda e: (e, 0, 0)),
        ],
        out_specs=pl.BlockSpec((CAP, DP), lambda e: (e, 0)),
        out_shape=jax.ShapeDtypeStruct((NROWS, DP), jnp.float32),
    )(xe, w1, b1, w2, b2)


def _combine(ye, idx):
    idx2 = idx.reshape(T // WIN, WIN)
    mesh = plsc.VectorSubcoreMesh(core_axis_name="core",
                                  subcore_axis_name="subcore")

    @functools.partial(
        pl.kernel,
        out_type=jax.ShapeDtypeStruct((T, DP), jnp.float32),
        mesh=mesh,
        scratch_types=[
            pltpu.VMEM((NCHUNK, WIN), jnp.int32),
            pltpu.VMEM((WIN, DP), jnp.float32),
            pltpu.VMEM((WIN, DP), jnp.float32),
            pltpu.SemaphoreType.DMA,
            pltpu.SemaphoreType.DMA,
            pltpu.SemaphoreType.DMA,
            pltpu.SemaphoreType.DMA,
        ])
    def k(ye_hbm, i_hbm, o_hbm, idx_v, buf0, buf1, sg0, sg1, st0, st1):
        wid = lax.axis_index("subcore") * NC + lax.axis_index("core")
        base = wid * CPT
        bufs, sgs, sts = (buf0, buf1), (sg0, sg1), (st0, st1)
        pltpu.sync_copy(i_hbm.at[pl.ds(wid * NCHUNK, NCHUNK)], idx_v)
        gats = [None] * NCHUNK
        stos = [None] * NCHUNK
        gats[0] = pltpu.async_copy(ye_hbm.at[idx_v.at[0]], bufs[0], sgs[0])
        for j in range(NCHUNK):
            b = j & 1
            gats[j].wait()
            stos[j] = pltpu.async_copy(
                bufs[b], o_hbm.at[pl.ds(base + j * WIN, WIN)], sts[b])
            if j >= 1:
                stos[j - 1].wait()
            if j + 1 < NCHUNK:
                gats[j + 1] = pltpu.async_copy(
                    ye_hbm.at[idx_v.at[j + 1]], bufs[(j + 1) & 1],
                    sgs[(j + 1) & 1])
        stos[NCHUNK - 1].wait()

    return k(ye, idx2)


def _final_body(x_ref, yg_ref, gk_ref, y_ref):
    gk = gk_ref[...]
    ya, yb = _unpack2(yg_ref[...])
    yg = jnp.concatenate([ya, yb], axis=1).astype(jnp.float32)
    y_ref[...] = x_ref[...] + jnp.where(gk > 0.0, gk * yg, 0.0)


def _final(x, yg, gk):
    return pl.pallas_call(
        _final_body,
        grid=(T // CHUNK,),
        in_specs=[
            pl.BlockSpec((CHUNK, D), lambda i: (i, 0)),
            pl.BlockSpec((CHUNK, DP), lambda i: (i, 0)),
            pl.BlockSpec((CHUNK, 1), lambda i: (i, 0)),
        ],
        out_specs=pl.BlockSpec((CHUNK, D), lambda i: (i, 0)),
        out_shape=jax.ShapeDtypeStruct((T, D), jnp.float32),
    )(x, yg, gk)


def kernel(x, Wg, W1, b1, W2, b2):
    idxd, idxc, gk, xbf = _router(x, Wg)
    xe = _dispatch(xbf, idxd.reshape(T))
    ye = _ffn(xe, W1, b1.reshape(E, 1, FF), W2, b2.reshape(E, 1, D))
    yg = _combine(ye, idxc.reshape(T))
    return _final(x, yg, gk)


# speedup vs baseline: 1.6036x; 1.0140x over previous
import functools

import jax
import jax.numpy as jnp
from jax import lax
from jax.experimental import pallas as pl
from jax.experimental.pallas import tpu as pltpu
from jax.experimental.pallas import tpu_sc as plsc

E = 64
D = 1024
FF = 1024
T = 8192
CAP = 160
CHUNK = 1024
NROWS = E * CAP
XE_ROWS = NROWS + 8
WIN = 64


def _router_body(x_ref, wg_ref, idxd_ref, idxc_ref, gk_ref, xb_ref, cnt_ref):
    step = pl.program_id(0)

    @pl.when(step == 0)
    def _():
        cnt_ref[...] = jnp.zeros_like(cnt_ref)

    xb = x_ref[...].astype(jnp.bfloat16)
    xb_ref[...] = _pack2(xb[:, :DP], xb[:, DP:])
    wg = wg_ref[...]
    logits = jnp.dot(xb, wg.astype(jnp.bfloat16),
                     preferred_element_type=jnp.float32)
    m = jnp.max(logits, axis=1, keepdims=True)
    ssum = jnp.sum(jnp.exp(logits - m), axis=1, keepdims=True)
    g = 1.0 / ssum

    lane = lax.broadcasted_iota(jnp.int32, (CHUNK, E), 1)
    e_idx = jnp.min(jnp.where(logits == m, lane, E), axis=1, keepdims=True)
    onehot = (lane == e_idx).astype(jnp.float32)

    r = lax.broadcasted_iota(jnp.int32, (CHUNK, CHUNK), 0)
    c = lax.broadcasted_iota(jnp.int32, (CHUNK, CHUNK), 1)
    tri = (c < r).astype(jnp.bfloat16)
    pos_in = jnp.dot(tri, onehot.astype(jnp.bfloat16),
                     preferred_element_type=jnp.float32)

    carry = cnt_ref[0:1, :]
    pos = jnp.sum((pos_in + carry) * onehot, axis=1, keepdims=True)
    cnt_ref[0:1, :] = carry + jnp.sum(onehot, axis=0, keepdims=True)

    pos_i = pos.astype(jnp.int32)
    keep = pos_i < CAP
    slot = e_idx * CAP + jnp.minimum(pos_i, CAP - 1)
    idxd_ref[...] = jnp.where(keep, slot, NROWS)
    idxc_ref[...] = jnp.where(keep, slot, 0)
    gk_ref[...] = jnp.where(keep, g, 0.0)


def _router(x, wg):
    return pl.pallas_call(
        _router_body,
        grid=(T // CHUNK,),
        in_specs=[
            pl.BlockSpec((CHUNK, D), lambda i: (i, 0)),
            pl.BlockSpec((D, E), lambda i: (0, 0)),
        ],
        out_specs=[
            pl.BlockSpec((CHUNK, 1), lambda i: (i, 0)),
            pl.BlockSpec((CHUNK, 1), lambda i: (i, 0)),
            pl.BlockSpec((CHUNK, 1), lambda i: (i, 0)),
            pl.BlockSpec((CHUNK, DP), lambda i: (i, 0)),
        ],
        out_shape=[
            jax.ShapeDtypeStruct((T, 1), jnp.int32),
            jax.ShapeDtypeStruct((T, 1), jnp.int32),
            jax.ShapeDtypeStruct((T, 1), jnp.float32),
            jax.ShapeDtypeStruct((T, DP), jnp.float32),
        ],
        scratch_shapes=[pltpu.VMEM((8, E), jnp.float32)],
    )(x, wg)


DP = D // 2


def _pack2(a, b):
    au = lax.bitcast_convert_type(a, jnp.uint16).astype(jnp.uint32)
    bu = lax.bitcast_convert_type(b, jnp.uint16).astype(jnp.uint32)
    return lax.bitcast_convert_type((au << 16) | bu, jnp.float32)


def _unpack2(w):
    wu = lax.bitcast_convert_type(w, jnp.uint32)
    a = lax.bitcast_convert_type((wu >> 16).astype(jnp.uint16), jnp.bfloat16)
    b = lax.bitcast_convert_type(wu.astype(jnp.uint16), jnp.bfloat16)
    return a, b


NC = 2
NS = 16
NW = NC * NS
CPT = T // NW
NCHUNK = CPT // WIN


def _dispatch(x, idx):
    idx2 = idx.reshape(T // WIN, WIN)
    mesh = plsc.VectorSubcoreMesh(core_axis_name="core",
                                  subcore_axis_name="subcore")

    @functools.partial(
        pl.kernel,
        out_type=jax.ShapeDtypeStruct((XE_ROWS, DP), jnp.float32),
        mesh=mesh,
        scratch_types=[
            pltpu.VMEM((NCHUNK, WIN), jnp.int32),
            pltpu.VMEM((WIN, DP), jnp.float32),
            pltpu.VMEM((WIN, DP), jnp.float32),
            pltpu.SemaphoreType.DMA,
            pltpu.SemaphoreType.DMA,
            pltpu.SemaphoreType.DMA,
            pltpu.SemaphoreType.DMA,
        ])
    def k(x_hbm, i_hbm, o_hbm, idx_v, buf0, buf1, sl0, sl1, ss0, ss1):
        wid = lax.axis_index("subcore") * NC + lax.axis_index("core")
        base = wid * CPT
        bufs, sls, sss = (buf0, buf1), (sl0, sl1), (ss0, ss1)
        pltpu.sync_copy(i_hbm.at[pl.ds(wid * NCHUNK, NCHUNK)], idx_v)
        loads = [None] * NCHUNK
        scats = [None] * NCHUNK
        loads[0] = pltpu.async_copy(
            x_hbm.at[pl.ds(base, WIN)], bufs[0], sls[0])
        for j in range(NCHUNK):
            b = j & 1
            loads[j].wait()
            scats[j] = pltpu.async_copy(
                bufs[b], o_hbm.at[idx_v.at[j]], sss[b])
            if j >= 1:
                scats[j - 1].wait()
            if j + 1 < NCHUNK:
                loads[j + 1] = pltpu.async_copy(
                    x_hbm.at[pl.ds(base + (j + 1) * WIN, WIN)],
                    bufs[(j + 1) & 1], sls[(j + 1) & 1])
        scats[NCHUNK - 1].wait()

    return k(x, idx2)


def _ffn_body(xe_ref, w1_ref, b1_ref, w2_ref, b2_ref, ye_ref):
    xa, xc = _unpack2(xe_ref[...])
    xb = jnp.concatenate([xa, xc], axis=1)
    w1 = w1_ref[0].astype(jnp.bfloat16)
    h = jnp.maximum(
        jnp.dot(xb, w1, preferred_element_type=jnp.float32) + b1_ref[0], 0.0)
    w2 = w2_ref[0].astype(jnp.bfloat16)
    y = jnp.dot(h.astype(jnp.bfloat16), w2,
                preferred_element_type=jnp.float32) + b2_ref[0]
    yb = y.astype(jnp.bfloat16)
    ye_ref[...] = _pack2(yb[:, :DP], yb[:, DP:])


def _ffn(xe, w1, b1, w2, b2):
    return pl.pallas_call(
        _ffn_body,
        grid=(E,),
        in_specs=[
            pl.BlockSpec((CAP, DP), lambda e: (e, 0)),
            pl.BlockSpec((1, D, FF), lambda e: (e, 0, 0)),
            pl.BlockSpec((1, 1, FF), lambda e: (e, 0, 0)),
            pl.BlockSpec((1, FF, D), lambda e: (e, 0, 0)),
            pl.BlockSpec((1, 1, D), lambda e: (e, 0, 0)),
        ],
        out_specs=pl.BlockSpec((CAP, DP), lambda e: (e, 0)),
        out_shape=jax.ShapeDtypeStruct((NROWS, DP), jnp.float32),
    )(xe, w1, b1, w2, b2)


def _combine(ye, idx):
    idx2 = idx.reshape(T // WIN, WIN)
    mesh = plsc.VectorSubcoreMesh(core_axis_name="core",
                                  subcore_axis_name="subcore")

    @functools.partial(
        pl.kernel,
        out_type=jax.ShapeDtypeStruct((T, DP), jnp.float32),
        mesh=mesh,
        scratch_types=[
            pltpu.VMEM((NCHUNK, WIN), jnp.int32),
            pltpu.VMEM((WIN, DP), jnp.float32),
            pltpu.VMEM((WIN, DP), jnp.float32),
            pltpu.SemaphoreType.DMA,
            pltpu.SemaphoreType.DMA,
            pltpu.SemaphoreType.DMA,
            pltpu.SemaphoreType.DMA,
        ])
    def k(ye_hbm, i_hbm, o_hbm, idx_v, buf0, buf1, sg0, sg1, st0, st1):
        wid = lax.axis_index("subcore") * NC + lax.axis_index("core")
        base = wid * CPT
        bufs, sgs, sts = (buf0, buf1), (sg0, sg1), (st0, st1)
        pltpu.sync_copy(i_hbm.at[pl.ds(wid * NCHUNK, NCHUNK)], idx_v)
        gats = [None] * NCHUNK
        stos = [None] * NCHUNK
        gats[0] = pltpu.async_copy(ye_hbm.at[idx_v.at[0]], bufs[0], sgs[0])
        for j in range(NCHUNK):
            b = j & 1
            gats[j].wait()
            stos[j] = pltpu.async_copy(
                bufs[b], o_hbm.at[pl.ds(base + j * WIN, WIN)], sts[b])
            if j >= 1:
                stos[j - 1].wait()
            if j + 1 < NCHUNK:
                gats[j + 1] = pltpu.async_copy(
                    ye_hbm.at[idx_v.at[j + 1]], bufs[(j + 1) & 1],
                    sgs[(j + 1) & 1])
        stos[NCHUNK - 1].wait()

    return k(ye, idx2)


def _final_body(x_ref, yg_ref, gk_ref, y_ref):
    gk = gk_ref[...]
    xa, xc = _unpack2(x_ref[...])
    x = jnp.concatenate([xa, xc], axis=1).astype(jnp.float32)
    ya, yb = _unpack2(yg_ref[...])
    yg = jnp.concatenate([ya, yb], axis=1).astype(jnp.float32)
    y_ref[...] = x + jnp.where(gk > 0.0, gk * yg, 0.0)


def _final(x, yg, gk):
    return pl.pallas_call(
        _final_body,
        grid=(T // CHUNK,),
        in_specs=[
            pl.BlockSpec((CHUNK, DP), lambda i: (i, 0)),
            pl.BlockSpec((CHUNK, DP), lambda i: (i, 0)),
            pl.BlockSpec((CHUNK, 1), lambda i: (i, 0)),
        ],
        out_specs=pl.BlockSpec((CHUNK, D), lambda i: (i, 0)),
        out_shape=jax.ShapeDtypeStruct((T, D), jnp.float32),
    )(x, yg, gk)


def kernel(x, Wg, W1, b1, W2, b2):
    idxd, idxc, gk, xbf = _router(x, Wg)
    xe = _dispatch(xbf, idxd.reshape(T))
    ye = _ffn(xe, W1, b1.reshape(E, 1, FF), W2, b2.reshape(E, 1, D))
    yg = _combine(ye, idxc.reshape(T))
    return _final(xbf, yg, gk)
